# baseline (speedup 1.0000x reference)
import functools

import jax
import jax.numpy as jnp
from jax import lax
from jax.experimental import pallas as pl
from jax.experimental.pallas import tpu as pltpu
from jax.experimental.pallas import tpu_sc as plsc


def _build_gather(num_batch: int, num_rows: int, hidden: int):
    info = plsc.get_sparse_core_info()
    nc, ns = info.num_cores, info.num_subcores
    nw = nc * ns
    assert num_rows % (8 * nw) == 0
    b_per_w = num_rows // nw

    mesh = plsc.VectorSubcoreMesh(core_axis_name="c", subcore_axis_name="s")

    @functools.partial(
        pl.kernel,
        mesh=mesh,
        out_type=jax.ShapeDtypeStruct((num_rows, num_batch, hidden), jnp.float32),
        scratch_types=[
            pltpu.VMEM((b_per_w,), jnp.int32),
            pltpu.VMEM((b_per_w, hidden), jnp.float32),
            pltpu.SemaphoreType.DMA,
            pltpu.SemaphoreType.DMA,
        ],
    )
    def gather_kernel(idx_hbm, table_hbm, out_hbm, idx_v, rows_v, gsem, ssem):
        wid = lax.axis_index("s") * nc + lax.axis_index("c")
        base = wid * b_per_w
        half = b_per_w // 2
        pltpu.sync_copy(idx_hbm.at[0, pl.ds(base, b_per_w)], idx_v)
        g0 = pltpu.async_copy(
            table_hbm.at[idx_v.at[pl.ds(0, half)]], rows_v.at[pl.ds(0, half)], gsem
        )
        g1 = pltpu.async_copy(
            table_hbm.at[idx_v.at[pl.ds(half, half)]],
            rows_v.at[pl.ds(half, half)],
            gsem,
        )
        g0.wait()
        s0 = pltpu.async_copy(
            rows_v.at[pl.ds(0, half)], out_hbm.at[pl.ds(base, half), 0], ssem
        )
        g1.wait()
        s1 = pltpu.async_copy(
            rows_v.at[pl.ds(half, half)],
            out_hbm.at[pl.ds(base + half, half), 0],
            ssem,
        )
        s0.wait()
        s1.wait()

    return gather_kernel


def kernel(position_ids, table):
    num_batch, num_rows = position_ids.shape
    hidden = table.shape[-1]
    ids = position_ids.astype(jnp.int32)
    return _build_gather(num_batch, num_rows, hidden)(ids, table)


# speedup vs baseline: 1.0224x; 1.0224x over previous
import functools

import jax
import jax.numpy as jnp
from jax import lax
from jax.experimental import pallas as pl
from jax.experimental.pallas import tpu as pltpu
from jax.experimental.pallas import tpu_sc as plsc


def _build_gather(num_batch: int, num_rows: int, hidden: int):
    info = plsc.get_sparse_core_info()
    nc, ns = info.num_cores, info.num_subcores
    nw = nc * ns
    assert num_rows % (8 * nw) == 0
    b_per_w = num_rows // nw

    mesh = plsc.VectorSubcoreMesh(core_axis_name="c", subcore_axis_name="s")

    @functools.partial(
        pl.kernel,
        mesh=mesh,
        out_type=jax.ShapeDtypeStruct((num_rows, num_batch, hidden), jnp.float32),
        scratch_types=[
            pltpu.VMEM((b_per_w,), jnp.int32),
            pltpu.VMEM((b_per_w, hidden), jnp.float32),
            pltpu.SemaphoreType.DMA,
        ],
    )
    def gather_kernel(idx_hbm, table_hbm, out_hbm, idx_v, rows_v, sem):
        wid = lax.axis_index("s") * nc + lax.axis_index("c")
        base = wid * b_per_w
        pltpu.sync_copy(idx_hbm.at[0, pl.ds(base, b_per_w)], idx_v)
        pltpu.async_copy(table_hbm.at[idx_v], rows_v, sem).wait()
        pltpu.sync_copy(rows_v, out_hbm.at[pl.ds(base, b_per_w), 0])

    return gather_kernel


def kernel(position_ids, table):
    num_batch, num_rows = position_ids.shape
    hidden = table.shape[-1]
    ids = position_ids.astype(jnp.int32)
    return _build_gather(num_batch, num_rows, hidden)(ids, table)
